# baseline (speedup 1.0000x reference)
import functools

import jax
import jax.numpy as jnp
from jax import lax
from jax.experimental import pallas as pl
from jax.experimental.pallas import tpu as pltpu
from jax.experimental.pallas import tpu_sc as plsc

DIM = 64
NC = 2
NS = 16
NW = NC * NS
BB = 128
NB = 2
LANES = 16


@functools.lru_cache(maxsize=None)
def _make_lookup(B, L, n_rows):
  assert B == NW * BB and L % NB == 0
  n_groups = L // NB
  dt_tiles = DIM // 8
  mesh = plsc.VectorSubcoreMesh(core_axis_name="c", subcore_axis_name="s")

  @functools.partial(
      pl.kernel,
      mesh=mesh,
      compiler_params=pltpu.CompilerParams(use_tc_tiling_on_sc=False, needs_layout_passes=False),
      out_type=jax.ShapeDtypeStruct((L, dt_tiles, NW, 8, BB), jnp.float32),
      scratch_types=[
          pltpu.VMEM((L, BB), jnp.int32),
          pltpu.VMEM((L, BB), jnp.int32),
          pltpu.VMEM((NB, BB, DIM), jnp.float32),
          pltpu.VMEM((NB, dt_tiles, 1, 8, BB), jnp.float32),
          pltpu.VMEM_SHARED((n_rows, DIM), jnp.float32),
          pltpu.VMEM_SHARED((n_rows, DIM), jnp.float32),
      ] + [pltpu.SemaphoreType.DMA] * NB,
  )
  def lookup(t_tab, s_tab, t_idx, s_idx, out, tiv, siv, bufs, obufs, t_sh,
             s_sh, *sems):
    sid = lax.axis_index("s")
    wid = sid * NC + lax.axis_index("c")

    @pl.when(sid == 0)
    def _():
      pltpu.sync_copy(t_tab, t_sh)
      pltpu.sync_copy(s_tab, s_sh)

    pltpu.sync_copy(t_idx.at[wid], tiv)
    pltpu.sync_copy(s_idx.at[wid], siv)
    plsc.subcore_barrier()

    def fire_t(c, b):
      pltpu.async_copy(t_sh.at[tiv.at[c]], bufs.at[b], sems[b])

    def wait_t(c, b):
      pltpu.make_async_copy(t_sh.at[tiv.at[c]], bufs.at[b], sems[b]).wait()

    def fire_s(c, b):
      pltpu.async_copy(s_sh.at[siv.at[c]], bufs.at[b], sems[b], add=True)

    def wait_s(c, b):
      pltpu.make_async_copy(s_sh.at[siv.at[c]], bufs.at[b], sems[b]).wait()

    def fire_out(c, b):
      pltpu.async_copy(obufs.at[b], out.at[c, :, pl.ds(wid, 1)], sems[b])

    def wait_out(c, b):
      pltpu.make_async_copy(obufs.at[b], out.at[c, :, pl.ds(wid, 1)],
                            sems[b]).wait()

    iota = lax.iota(jnp.int32, LANES)
    zeros = iota * 0
    dt_rows = [d0 // 8 + iota // 8 for d0 in range(0, DIM, LANES)]
    di_rows = [(d0 + iota) % 8 for d0 in range(0, DIM, LANES)]

    def transpose_slot(b):
      def blk(i, carry):
        for u in range(4):
          bb = i * 4 + u
          cols = zeros + bb
          for k in range(DIM // LANES):
            v = bufs[b, bb, pl.ds(k * LANES, LANES)]
            plsc.store_scatter(obufs.at[b], [dt_rows[k], zeros, di_rows[k], cols], v)
        return carry

      lax.fori_loop(0, BB // 4, blk, 0)

    for b in range(NB):
      fire_t(b, b)

    def group_body(g, carry):
      base = g * NB
      for b in range(NB):
        wait_t(base + b, b)
        fire_s(base + b, b)
      for b in range(NB):
        wait_s(base + b, b)
        transpose_slot(b)
        fire_out(base + b, b)
      for b in range(NB):
        wait_out(base + b, b)

        @pl.when(g < n_groups - 1)
        def _():
          fire_t(base + NB + b, b)

      return carry

    lax.fori_loop(0, n_groups, group_body, 0)

  return lookup


def kernel(times, spaces, time_emb, space_emb):
  B, L = times.shape
  t_idx = times.reshape(NW, BB, L).transpose(0, 2, 1).astype(jnp.int32)
  s_idx = spaces.reshape(NW, BB, L).transpose(0, 2, 1).astype(jnp.int32)
  out5 = _make_lookup(B, L, time_emb.shape[0])(time_emb, space_emb, t_idx,
                                               s_idx)
  return jnp.transpose(out5, (2, 4, 0, 1, 3)).reshape(B, L, DIM)


# speedup vs baseline: 1.3136x; 1.3136x over previous
import functools

import jax
import jax.numpy as jnp
from jax import lax
from jax.experimental import pallas as pl
from jax.experimental.pallas import tpu as pltpu
from jax.experimental.pallas import tpu_sc as plsc

DIM = 64
NC = 2
NS = 16
NW = NC * NS
BB = 128
NB = 2
LANES = 16


@functools.lru_cache(maxsize=None)
def _make_lookup(B, L, n_rows):
  assert B == NW * BB and L % NB == 0
  n_groups = L // NB
  dt_tiles = DIM // 8
  mesh = plsc.VectorSubcoreMesh(core_axis_name="c", subcore_axis_name="s")

  @functools.partial(
      pl.kernel,
      mesh=mesh,
      compiler_params=pltpu.CompilerParams(use_tc_tiling_on_sc=False, needs_layout_passes=False),
      out_type=jax.ShapeDtypeStruct((L, dt_tiles, NW, 8, BB), jnp.float32),
      scratch_types=[
          pltpu.VMEM((L, BB), jnp.int32),
          pltpu.VMEM((L, BB), jnp.int32),
          pltpu.VMEM((NB, BB, DIM), jnp.float32),
          pltpu.VMEM((NB, dt_tiles, 1, 8, BB), jnp.float32),
          pltpu.VMEM_SHARED((n_rows, DIM), jnp.float32),
          pltpu.VMEM_SHARED((n_rows, DIM), jnp.float32),
      ] + [pltpu.SemaphoreType.DMA] * NB,
  )
  def lookup(t_tab, s_tab, t_idx, s_idx, out, tiv, siv, bufs, obufs, t_sh,
             s_sh, *sems):
    sid = lax.axis_index("s")
    wid = sid * NC + lax.axis_index("c")

    @pl.when(sid == 0)
    def _():
      pltpu.sync_copy(t_tab, t_sh)
      pltpu.sync_copy(s_tab, s_sh)

    pltpu.sync_copy(t_idx.at[wid], tiv)
    pltpu.sync_copy(s_idx.at[wid], siv)
    plsc.subcore_barrier()

    def fire_t(c, b):
      pltpu.async_copy(t_sh.at[tiv.at[c]], bufs.at[b], sems[b])

    def wait_t(c, b):
      pltpu.make_async_copy(t_sh.at[tiv.at[c]], bufs.at[b], sems[b]).wait()

    def fire_s(c, b):
      pltpu.async_copy(s_sh.at[siv.at[c]], bufs.at[b], sems[b], add=True)

    def wait_s(c, b):
      pltpu.make_async_copy(s_sh.at[siv.at[c]], bufs.at[b], sems[b]).wait()

    def fire_out(c, b):
      pltpu.async_copy(obufs.at[b], out.at[c, :, pl.ds(wid, 1)], sems[b])

    def wait_out(c, b):
      pltpu.make_async_copy(obufs.at[b], out.at[c, :, pl.ds(wid, 1)],
                            sems[b]).wait()

    iota = lax.iota(jnp.int32, LANES)

    def transpose_slot(b):
      for i in range(BB // LANES):
        rows = iota + i * LANES
        for d0 in range(0, DIM, 32):
          vs = [
              plsc.load_gather(bufs.at[b],
                               [rows, jnp.full((LANES,), d0 + j, jnp.int32)])
              for j in range(32)
          ]
          for j in range(32):
            d = d0 + j
            obufs[b, d // 8, 0, d % 8, pl.ds(i * LANES, LANES)] = vs[j]

    for b in range(NB):
      fire_t(b, b)

    def group_body(g, carry):
      base = g * NB
      for b in range(NB):
        wait_t(base + b, b)
        fire_s(base + b, b)
      for b in range(NB):
        wait_s(base + b, b)
        transpose_slot(b)
        fire_out(base + b, b)
      for b in range(NB):
        wait_out(base + b, b)

        @pl.when(g < n_groups - 1)
        def _():
          fire_t(base + NB + b, b)

      return carry

    lax.fori_loop(0, n_groups, group_body, 0)

  return lookup


def kernel(times, spaces, time_emb, space_emb):
  B, L = times.shape
  t_idx = times.reshape(NW, BB, L).transpose(0, 2, 1).astype(jnp.int32)
  s_idx = spaces.reshape(NW, BB, L).transpose(0, 2, 1).astype(jnp.int32)
  out5 = _make_lookup(B, L, time_emb.shape[0])(time_emb, space_emb, t_idx,
                                               s_idx)
  return jnp.transpose(out5, (2, 4, 0, 1, 3)).reshape(B, L, DIM)


# speedup vs baseline: 1.5791x; 1.2021x over previous
import functools

import jax
import jax.numpy as jnp
from jax import lax
from jax.experimental import pallas as pl
from jax.experimental.pallas import tpu as pltpu
from jax.experimental.pallas import tpu_sc as plsc

DIM = 64
NC = 2
NS = 16
NW = NC * NS
NB = 4


@functools.lru_cache(maxsize=None)
def _make_lookup(B, L, n_rows):
  n_chunks = B // NW
  assert n_chunks % NB == 0
  n_groups = n_chunks // NB
  mesh = plsc.VectorSubcoreMesh(core_axis_name="c", subcore_axis_name="s")

  @functools.partial(
      pl.kernel,
      mesh=mesh,
      compiler_params=pltpu.CompilerParams(use_tc_tiling_on_sc=False),
      out_type=jax.ShapeDtypeStruct((B, L, DIM), jnp.float32),
      scratch_types=[
          pltpu.VMEM((n_chunks, L), jnp.int32),
          pltpu.VMEM((n_chunks, L), jnp.int32),
          pltpu.VMEM((NB, L, DIM), jnp.float32),
          pltpu.VMEM_SHARED((n_rows, DIM), jnp.float32),
          pltpu.VMEM_SHARED((n_rows, DIM), jnp.float32),
      ] + [pltpu.SemaphoreType.DMA] * NB,
  )
  def lookup(t_tab, s_tab, t_idx, s_idx, out, tiv, siv, bufs, t_sh, s_sh,
             *sems):
    sid = lax.axis_index("s")
    wid = sid * NC + lax.axis_index("c")

    @pl.when(sid == 0)
    def _():
      pltpu.sync_copy(t_tab, t_sh)
      pltpu.sync_copy(s_tab, s_sh)

    pltpu.sync_copy(t_idx.at[wid], tiv)
    pltpu.sync_copy(s_idx.at[wid], siv)
    plsc.subcore_barrier()

    def fire_t(c, b):
      pltpu.async_copy(t_sh.at[tiv.at[c]], bufs.at[b], sems[b])

    def wait_t(c, b):
      pltpu.make_async_copy(t_sh.at[tiv.at[c]], bufs.at[b], sems[b]).wait()

    def fire_s(c, b):
      pltpu.async_copy(s_sh.at[siv.at[c]], bufs.at[b], sems[b], add=True)

    def wait_s(c, b):
      pltpu.make_async_copy(s_sh.at[siv.at[c]], bufs.at[b], sems[b]).wait()

    def fire_out(c, b):
      pltpu.async_copy(bufs.at[b], out.at[wid * n_chunks + c], sems[b])

    def wait_out(c, b):
      pltpu.make_async_copy(bufs.at[b], out.at[wid * n_chunks + c],
                            sems[b]).wait()

    for b in range(NB):
      fire_t(b, b)

    def group_body(g, carry):
      base = g * NB
      for b in range(NB):
        wait_t(base + b, b)
        fire_s(base + b, b)
      for b in range(NB):
        wait_s(base + b, b)
        fire_out(base + b, b)
      for b in range(NB):
        wait_out(base + b, b)

        @pl.when(g < n_groups - 1)
        def _():
          fire_t(base + NB + b, b)

      return carry

    lax.fori_loop(0, n_groups, group_body, 0)

  return lookup


def kernel(times, spaces, time_emb, space_emb):
  B, L = times.shape
  assert B % NW == 0
  n_chunks = B // NW
  t_idx = times.reshape(NW, n_chunks, L).astype(jnp.int32)
  s_idx = spaces.reshape(NW, n_chunks, L).astype(jnp.int32)
  return _make_lookup(B, L, time_emb.shape[0])(time_emb, space_emb, t_idx,
                                               s_idx)
